# Initial kernel scaffold; baseline (speedup 1.0000x reference)
import jax
import jax.numpy as jnp
from jax.experimental import pallas as pl


def kernel(x, edge_index, W1, a1, W2, a2):
    raise NotImplementedError("write your pallas kernel here")


# speedup vs baseline: 52.6004x; 52.6004x over previous
import functools

import jax
import jax.numpy as jnp
from jax import lax
from jax.experimental import pallas as pl
from jax.experimental.pallas import tpu as pltpu
from jax.experimental.pallas import tpu_sc as plsc

_N = 10000
_E = 320000
_NFEAT = 128
_NHID = 16
_NHEADS = 8
_NCLASS = 32
_NEG = 0.2

_NC = 2
_NS = 16
_NW = _NC * _NS
_C = 80
_STEPS = 128
_EP = _NW * _STEPS * _C
_NP = 10240
_ROWS = _NP // _NS


def _tc1_body(x_ref, w_ref, asrc_ref, adst_ref, wh_ref, tabs_ref, tabd_ref):
    wh = jnp.dot(x_ref[...], w_ref[...], preferred_element_type=jnp.float32)
    wh_ref[pl.ds(0, _N), :] = wh
    wh_ref[pl.ds(_N, _NP - _N), :] = jnp.zeros((_NP - _N, _NFEAT), jnp.float32)
    ssrc = jnp.dot(wh, asrc_ref[...], preferred_element_type=jnp.float32)
    sdst = jnp.dot(wh, adst_ref[...], preferred_element_type=jnp.float32)
    tabs_ref[pl.ds(0, _N), :] = jnp.concatenate([ssrc, sdst], axis=1)
    tabs_ref[pl.ds(_N, _NP - _N), :] = jnp.zeros((_NP - _N, 2 * _NHEADS),
                                                 jnp.float32)
    tabd_ref[pl.ds(0, _N), :] = jnp.concatenate([sdst, ssrc], axis=1)
    tabd_ref[pl.ds(_N, _NP - _N), :] = jnp.zeros((_NP - _N, 2 * _NHEADS),
                                                 jnp.float32)


def _tc1(x, w1cat, asrc, adst):
    return pl.pallas_call(
        _tc1_body,
        out_shape=[
            jax.ShapeDtypeStruct((_NP, _NFEAT), jnp.float32),
            jax.ShapeDtypeStruct((_NP, 2 * _NHEADS), jnp.float32),
            jax.ShapeDtypeStruct((_NP, 2 * _NHEADS), jnp.float32),
        ],
    )(x, w1cat, asrc, adst)


def _sc1_body(src_hbm, dst_hbm, ssrc_hbm, sdst_hbm, wh_hbm, zzero_hbm, dzero_hbm,
              zpart_hbm, dpart_hbm,
              srcv, dstv, sbuf, dbuf, pbuf, fbuf, zsh, dsh, fsem):
    cid = lax.axis_index("c")
    sid = lax.axis_index("s")
    wid = sid * _NC + cid
    r0 = sid * _ROWS

    pltpu.sync_copy(zzero_hbm.at[pl.ds(r0, _ROWS), :], zsh.at[pl.ds(r0, _ROWS), :])
    pltpu.sync_copy(dzero_hbm.at[pl.ds(r0, _ROWS), :], dsh.at[pl.ds(r0, _ROWS), :])

    pltpu.sync_copy(src_hbm.at[pl.ds(wid * _STEPS, _STEPS), :], srcv)
    pltpu.sync_copy(dst_hbm.at[pl.ds(wid * _STEPS, _STEPS), :], dstv)
    plsc.subcore_barrier()

    def step(s, carry):
        sidx = srcv.at[s]
        didx = dstv.at[s]
        pltpu.sync_copy(ssrc_hbm.at[sidx], sbuf)
        pltpu.sync_copy(sdst_hbm.at[didx], dbuf)
        fcp = pltpu.async_copy(wh_hbm.at[sidx], fbuf, fsem)

        def pstep(e, c):
            t = sbuf[e, :] + dbuf[e, :]
            t = jnp.maximum(t, t * _NEG)
            pbuf[e, :] = jnp.exp(t)
            return c
        lax.fori_loop(0, _C, pstep, 0)

        fcp.wait()

        def escale(e, c):
            pv = pbuf[e, :]
            for h in range(_NHEADS):
                seg = fbuf[e, pl.ds(h * _NHID, _NHID)]
                fbuf[e, pl.ds(h * _NHID, _NHID)] = seg * pv[h]
            return c
        lax.fori_loop(0, _C, escale, 0)

        pltpu.sync_copy(pbuf, dsh.at[didx], add=True)
        pltpu.sync_copy(fbuf, zsh.at[didx], add=True)
        return carry

    lax.fori_loop(0, _STEPS, step, 0)

    plsc.subcore_barrier()
    pltpu.sync_copy(zsh.at[pl.ds(r0, _ROWS), :], zpart_hbm.at[cid, pl.ds(r0, _ROWS), :])
    pltpu.sync_copy(dsh.at[pl.ds(r0, _ROWS), :], dpart_hbm.at[cid, pl.ds(r0, _ROWS), :])


def _sc1(src2d, dst2d, ssrc, sdst, wh, zzero, dzero):
    mesh = plsc.VectorSubcoreMesh(
        core_axis_name="c", subcore_axis_name="s", num_cores=_NC, num_subcores=_NS)
    f = functools.partial(
        pl.kernel,
        out_type=[
            jax.ShapeDtypeStruct((_NC, _NP, _NFEAT), jnp.float32),
            jax.ShapeDtypeStruct((_NC, _NP, 2 * _NHEADS), jnp.float32),
        ],
        mesh=mesh,
        scratch_types=[
            pltpu.VMEM((_STEPS, _C), jnp.int32),
            pltpu.VMEM((_STEPS, _C), jnp.int32),
            pltpu.VMEM((_C, 2 * _NHEADS), jnp.float32),
            pltpu.VMEM((_C, 2 * _NHEADS), jnp.float32),
            pltpu.VMEM((_C, 2 * _NHEADS), jnp.float32),
            pltpu.VMEM((_C, _NFEAT), jnp.float32),
            pltpu.VMEM_SHARED((_NP, _NFEAT), jnp.float32),
            pltpu.VMEM_SHARED((_NP, 2 * _NHEADS), jnp.float32),
            pltpu.SemaphoreType.DMA,
        ],
        compiler_params=pltpu.CompilerParams(use_tc_tiling_on_sc=False),
    )(_sc1_body)
    return f(src2d, dst2d, ssrc, sdst, wh, zzero, dzero)


def _tc2_body(z_ref, d_ref, w2_ref, asrc_ref, adst_ref,
              wh2_ref, tabs2_ref, tabd2_ref):
    z = z_ref[0] + z_ref[1]
    d = jnp.maximum(d_ref[0, :, :_NHEADS] + d_ref[1, :, :_NHEADS], 1e-16)
    pieces = [z[:, h * _NHID:(h + 1) * _NHID] / d[:, h:h + 1]
              for h in range(_NHEADS)]
    o = jnp.concatenate(pieces, axis=1)
    h1 = jnp.where(o > 0, o, jnp.exp(jnp.minimum(o, 0.0)) - 1.0)
    wh2 = jnp.dot(h1, w2_ref[...], preferred_element_type=jnp.float32)
    wh2_ref[...] = wh2
    ssrc2 = jnp.dot(wh2, asrc_ref[...], preferred_element_type=jnp.float32)
    sdst2 = jnp.dot(wh2, adst_ref[...], preferred_element_type=jnp.float32)
    pad = jnp.zeros((wh2.shape[0], 15), jnp.float32)
    tabs2_ref[...] = jnp.concatenate([ssrc2, pad], axis=1)
    tabd2_ref[...] = jnp.concatenate([sdst2, pad], axis=1)


_BR = 2048


def _tc2(zpart, dpart, w2, a2src, a2dst):
    return pl.pallas_call(
        _tc2_body,
        grid=(_NP // _BR,),
        in_specs=[
            pl.BlockSpec((_NC, _BR, _NFEAT), lambda i: (0, i, 0)),
            pl.BlockSpec((_NC, _BR, 2 * _NHEADS), lambda i: (0, i, 0)),
            pl.BlockSpec((_NFEAT, _NCLASS), lambda i: (0, 0)),
            pl.BlockSpec((_NCLASS, 1), lambda i: (0, 0)),
            pl.BlockSpec((_NCLASS, 1), lambda i: (0, 0)),
        ],
        out_specs=[
            pl.BlockSpec((_BR, _NCLASS), lambda i: (i, 0)),
            pl.BlockSpec((_BR, 16), lambda i: (i, 0)),
            pl.BlockSpec((_BR, 16), lambda i: (i, 0)),
        ],
        out_shape=[
            jax.ShapeDtypeStruct((_NP, _NCLASS), jnp.float32),
            jax.ShapeDtypeStruct((_NP, 16), jnp.float32),
            jax.ShapeDtypeStruct((_NP, 16), jnp.float32),
        ],
    )(zpart, dpart, w2, a2src, a2dst)


def _sc2_body(src_hbm, dst_hbm, ssrc_hbm, sdst_hbm, wh_hbm, zzero_hbm, dzero_hbm,
              zpart_hbm, dpart_hbm,
              srcv, dstv, sbuf, dbuf, pbuf, fbuf, zsh, dsh, fsem):
    cid = lax.axis_index("c")
    sid = lax.axis_index("s")
    wid = sid * _NC + cid
    r0 = sid * _ROWS

    pltpu.sync_copy(zzero_hbm.at[pl.ds(r0, _ROWS), :], zsh.at[pl.ds(r0, _ROWS), :])
    pltpu.sync_copy(dzero_hbm.at[pl.ds(r0, _ROWS), :], dsh.at[pl.ds(r0, _ROWS), :])

    pltpu.sync_copy(src_hbm.at[pl.ds(wid * _STEPS, _STEPS), :], srcv)
    pltpu.sync_copy(dst_hbm.at[pl.ds(wid * _STEPS, _STEPS), :], dstv)
    plsc.subcore_barrier()

    def step(s, carry):
        sidx = srcv.at[s]
        didx = dstv.at[s]
        pltpu.sync_copy(ssrc_hbm.at[sidx], sbuf)
        pltpu.sync_copy(sdst_hbm.at[didx], dbuf)
        fcp = pltpu.async_copy(wh_hbm.at[sidx], fbuf, fsem)

        def pstep(e, c):
            t = sbuf[e, :] + dbuf[e, :]
            t = jnp.maximum(t, t * _NEG)
            pbuf[e, :] = jnp.exp(t)
            return c
        lax.fori_loop(0, _C, pstep, 0)

        fcp.wait()

        def escale(e, c):
            pv = pbuf[e, :]
            ph = pv[0]
            fbuf[e, pl.ds(0, 16)] = fbuf[e, pl.ds(0, 16)] * ph
            fbuf[e, pl.ds(16, 16)] = fbuf[e, pl.ds(16, 16)] * ph
            return c
        lax.fori_loop(0, _C, escale, 0)

        pltpu.sync_copy(pbuf, dsh.at[didx], add=True)
        pltpu.sync_copy(fbuf, zsh.at[didx], add=True)
        return carry

    lax.fori_loop(0, _STEPS, step, 0)

    plsc.subcore_barrier()
    pltpu.sync_copy(zsh.at[pl.ds(r0, _ROWS), :], zpart_hbm.at[cid, pl.ds(r0, _ROWS), :])
    pltpu.sync_copy(dsh.at[pl.ds(r0, _ROWS), :], dpart_hbm.at[cid, pl.ds(r0, _ROWS), :])


def _sc2(src2d, dst2d, tabs2, tabd2, wh2, z2zero, d2zero):
    mesh = plsc.VectorSubcoreMesh(
        core_axis_name="c", subcore_axis_name="s", num_cores=_NC, num_subcores=_NS)
    f = functools.partial(
        pl.kernel,
        out_type=[
            jax.ShapeDtypeStruct((_NC, _NP, _NCLASS), jnp.float32),
            jax.ShapeDtypeStruct((_NC, _NP, 16), jnp.float32),
        ],
        mesh=mesh,
        scratch_types=[
            pltpu.VMEM((_STEPS, _C), jnp.int32),
            pltpu.VMEM((_STEPS, _C), jnp.int32),
            pltpu.VMEM((_C, 16), jnp.float32),
            pltpu.VMEM((_C, 16), jnp.float32),
            pltpu.VMEM((_C, 16), jnp.float32),
            pltpu.VMEM((_C, _NCLASS), jnp.float32),
            pltpu.VMEM_SHARED((_NP, _NCLASS), jnp.float32),
            pltpu.VMEM_SHARED((_NP, 16), jnp.float32),
            pltpu.SemaphoreType.DMA,
        ],
        compiler_params=pltpu.CompilerParams(use_tc_tiling_on_sc=False),
    )(_sc2_body)
    return f(src2d, dst2d, tabs2, tabd2, wh2, z2zero, d2zero)


def _tc3_body(z_ref, d_ref, out_ref):
    z = z_ref[0] + z_ref[1]
    d = jnp.maximum(d_ref[0, :, :1] + d_ref[1, :, :1], 1e-16)
    out_ref[...] = z / d


def _tc3(z2part, d2part):
    return pl.pallas_call(
        _tc3_body,
        out_shape=jax.ShapeDtypeStruct((_NP, _NCLASS), jnp.float32),
    )(z2part, d2part)


def kernel(x, edge_index, W1, a1, W2, a2):
    w1cat = W1.transpose(1, 0, 2).reshape(_NFEAT, _NHEADS * _NHID)
    rows = jnp.arange(_NHEADS * _NHID)
    asrc = jnp.zeros((_NHEADS * _NHID, _NHEADS), jnp.float32).at[
        rows, rows // _NHID].set(a1[:, _NHID:].reshape(-1))
    adst = jnp.zeros((_NHEADS * _NHID, _NHEADS), jnp.float32).at[
        rows, rows // _NHID].set(a1[:, :_NHID].reshape(-1))
    a2src = a2[_NCLASS:].reshape(_NCLASS, 1)
    a2dst = a2[:_NCLASS].reshape(_NCLASS, 1)

    pad_idx = (_N + jnp.arange(_EP - _E, dtype=jnp.int32) % (_NP - _N))
    src2d = jnp.concatenate(
        [edge_index[0].astype(jnp.int32), pad_idx]).reshape(_EP // _C, _C)
    dst2d = jnp.concatenate(
        [edge_index[1].astype(jnp.int32), pad_idx]).reshape(_EP // _C, _C)

    zzero = jnp.zeros((_NP, _NFEAT), jnp.float32)
    dzero = jnp.zeros((_NP, 2 * _NHEADS), jnp.float32)
    z2zero = jnp.zeros((_NP, _NCLASS), jnp.float32)
    d2zero = jnp.zeros((_NP, 16), jnp.float32)

    wh1, tabs1, tabd1 = _tc1(x, w1cat, asrc, adst)
    zpart, dpart = _sc1(src2d, dst2d, tabs1, tabd1, wh1, zzero, dzero)
    wh2, tabs2, tabd2 = _tc2(zpart, dpart, W2, a2src, a2dst)
    z2part, d2part = _sc2(src2d, dst2d, tabs2, tabd2, wh2, z2zero, d2zero)
    out = _tc3(z2part, d2part)
    return out[:_N]


# speedup vs baseline: 73.5685x; 1.3986x over previous
import functools

import jax
import jax.numpy as jnp
from jax import lax
from jax.experimental import pallas as pl
from jax.experimental.pallas import tpu as pltpu
from jax.experimental.pallas import tpu_sc as plsc

_N = 10000
_E = 320000
_NFEAT = 128
_NHID = 16
_NHEADS = 8
_NCLASS = 32
_NEG = 0.2

_NC = 2
_NS = 16
_NW = _NC * _NS
_C = 40
_STEPS = 256
_EP = _NW * _STEPS * _C
_NP = 10240
_ROWS = _NP // _NS


def _tc1_body(x_ref, w_ref, asrc_ref, adst_ref, wh_ref, tabs_ref, tabd_ref):
    wh = jnp.dot(x_ref[...], w_ref[...], preferred_element_type=jnp.float32)
    wh_ref[pl.ds(0, _N), :] = wh
    wh_ref[pl.ds(_N, _NP - _N), :] = jnp.zeros((_NP - _N, _NFEAT), jnp.float32)
    ssrc = jnp.dot(wh, asrc_ref[...], preferred_element_type=jnp.float32)
    sdst = jnp.dot(wh, adst_ref[...], preferred_element_type=jnp.float32)
    tabs_ref[pl.ds(0, _N), :] = jnp.concatenate([ssrc, sdst], axis=1)
    tabs_ref[pl.ds(_N, _NP - _N), :] = jnp.zeros((_NP - _N, 2 * _NHEADS),
                                                 jnp.float32)
    tabd_ref[pl.ds(0, _N), :] = jnp.concatenate([sdst, ssrc], axis=1)
    tabd_ref[pl.ds(_N, _NP - _N), :] = jnp.zeros((_NP - _N, 2 * _NHEADS),
                                                 jnp.float32)


def _tc1(x, w1cat, asrc, adst):
    return pl.pallas_call(
        _tc1_body,
        out_shape=[
            jax.ShapeDtypeStruct((_NP, _NFEAT), jnp.float32),
            jax.ShapeDtypeStruct((_NP, 2 * _NHEADS), jnp.float32),
            jax.ShapeDtypeStruct((_NP, 2 * _NHEADS), jnp.float32),
        ],
    )(x, w1cat, asrc, adst)


def _sc1_body(src_hbm, dst_hbm, ssrc_hbm, sdst_hbm, wh_hbm, zzero_hbm, dzero_hbm,
              zpart_hbm, dpart_hbm,
              srcv, dstv,
              sbuf0, dbuf0, pbuf0, fbuf0, sbuf1, dbuf1, pbuf1, fbuf1,
              zsh, dsh, gsem0, gsem1, csem0, csem1):
    cid = lax.axis_index("c")
    sid = lax.axis_index("s")
    wid = sid * _NC + cid
    r0 = sid * _ROWS

    pltpu.sync_copy(zzero_hbm.at[pl.ds(r0, _ROWS), :], zsh.at[pl.ds(r0, _ROWS), :])
    pltpu.sync_copy(dzero_hbm.at[pl.ds(r0, _ROWS), :], dsh.at[pl.ds(r0, _ROWS), :])

    pltpu.sync_copy(src_hbm.at[pl.ds(wid * _STEPS, _STEPS), :], srcv)
    pltpu.sync_copy(dst_hbm.at[pl.ds(wid * _STEPS, _STEPS), :], dstv)
    plsc.subcore_barrier()

    bufs = ((sbuf0, dbuf0, pbuf0, fbuf0, gsem0, csem0),
            (sbuf1, dbuf1, pbuf1, fbuf1, gsem1, csem1))

    def issue_gathers(s, b):
        sb, db, _, fb, gsem, _ = bufs[b]
        pltpu.async_copy(ssrc_hbm.at[srcv.at[s]], sb, gsem)
        pltpu.async_copy(sdst_hbm.at[dstv.at[s]], db, gsem)
        pltpu.async_copy(wh_hbm.at[srcv.at[s]], fb, gsem)

    def do_step(s, b):
        sb, db, pb, fb, gsem, csem = bufs[b]
        pltpu.make_async_copy(ssrc_hbm.at[srcv.at[s]], sb, gsem).wait()
        pltpu.make_async_copy(sdst_hbm.at[dstv.at[s]], db, gsem).wait()
        pltpu.make_async_copy(wh_hbm.at[srcv.at[s]], fb, gsem).wait()

        def fuse(e, c):
            t = sb[e, :] + db[e, :]
            t = jnp.maximum(t, t * _NEG)
            pv = jnp.exp(t)
            pb[e, :] = pv
            for h in range(_NHEADS):
                seg = fb[e, pl.ds(h * _NHID, _NHID)]
                fb[e, pl.ds(h * _NHID, _NHID)] = seg * pv[h]
            return c
        lax.fori_loop(0, _C, fuse, 0)

        c1 = pltpu.async_copy(pb, dsh.at[dstv.at[s]], csem, add=True)
        c2 = pltpu.async_copy(fb, zsh.at[dstv.at[s]], csem, add=True)
        c1.wait()
        c2.wait()

        @pl.when(s < _STEPS - 2)
        def _():
            issue_gathers(s + 2, b)

    issue_gathers(0, 0)
    issue_gathers(1, 1)

    def pair(i, carry):
        do_step(2 * i, 0)
        do_step(2 * i + 1, 1)
        return carry
    lax.fori_loop(0, _STEPS // 2, pair, 0)

    plsc.subcore_barrier()
    pltpu.sync_copy(zsh.at[pl.ds(r0, _ROWS), :], zpart_hbm.at[cid, pl.ds(r0, _ROWS), :])
    pltpu.sync_copy(dsh.at[pl.ds(r0, _ROWS), :], dpart_hbm.at[cid, pl.ds(r0, _ROWS), :])


def _sc1(src2d, dst2d, ssrc, sdst, wh, zzero, dzero):
    mesh = plsc.VectorSubcoreMesh(
        core_axis_name="c", subcore_axis_name="s", num_cores=_NC, num_subcores=_NS)
    f = functools.partial(
        pl.kernel,
        out_type=[
            jax.ShapeDtypeStruct((_NC, _NP, _NFEAT), jnp.float32),
            jax.ShapeDtypeStruct((_NC, _NP, 2 * _NHEADS), jnp.float32),
        ],
        mesh=mesh,
        scratch_types=[
            pltpu.VMEM((_STEPS, _C), jnp.int32),
            pltpu.VMEM((_STEPS, _C), jnp.int32),
            pltpu.VMEM((_C, 2 * _NHEADS), jnp.float32),
            pltpu.VMEM((_C, 2 * _NHEADS), jnp.float32),
            pltpu.VMEM((_C, 2 * _NHEADS), jnp.float32),
            pltpu.VMEM((_C, _NFEAT), jnp.float32),
            pltpu.VMEM((_C, 2 * _NHEADS), jnp.float32),
            pltpu.VMEM((_C, 2 * _NHEADS), jnp.float32),
            pltpu.VMEM((_C, 2 * _NHEADS), jnp.float32),
            pltpu.VMEM((_C, _NFEAT), jnp.float32),
            pltpu.VMEM_SHARED((_NP, _NFEAT), jnp.float32),
            pltpu.VMEM_SHARED((_NP, 2 * _NHEADS), jnp.float32),
            pltpu.SemaphoreType.DMA,
            pltpu.SemaphoreType.DMA,
            pltpu.SemaphoreType.DMA,
            pltpu.SemaphoreType.DMA,
        ],
        compiler_params=pltpu.CompilerParams(use_tc_tiling_on_sc=False),
    )(_sc1_body)
    return f(src2d, dst2d, ssrc, sdst, wh, zzero, dzero)


def _tc2_body(z_ref, d_ref, w2_ref, asrc_ref, adst_ref,
              wh2_ref, tabs2_ref, tabd2_ref):
    z = z_ref[0] + z_ref[1]
    d = jnp.maximum(d_ref[0, :, :_NHEADS] + d_ref[1, :, :_NHEADS], 1e-16)
    pieces = [z[:, h * _NHID:(h + 1) * _NHID] / d[:, h:h + 1]
              for h in range(_NHEADS)]
    o = jnp.concatenate(pieces, axis=1)
    h1 = jnp.where(o > 0, o, jnp.exp(jnp.minimum(o, 0.0)) - 1.0)
    wh2 = jnp.dot(h1, w2_ref[...], preferred_element_type=jnp.float32)
    wh2_ref[...] = wh2
    ssrc2 = jnp.dot(wh2, asrc_ref[...], preferred_element_type=jnp.float32)
    sdst2 = jnp.dot(wh2, adst_ref[...], preferred_element_type=jnp.float32)
    pad = jnp.zeros((wh2.shape[0], 15), jnp.float32)
    tabs2_ref[...] = jnp.concatenate([ssrc2, pad], axis=1)
    tabd2_ref[...] = jnp.concatenate([sdst2, pad], axis=1)


_BR = 2048


def _tc2(zpart, dpart, w2, a2src, a2dst):
    return pl.pallas_call(
        _tc2_body,
        grid=(_NP // _BR,),
        in_specs=[
            pl.BlockSpec((_NC, _BR, _NFEAT), lambda i: (0, i, 0)),
            pl.BlockSpec((_NC, _BR, 2 * _NHEADS), lambda i: (0, i, 0)),
            pl.BlockSpec((_NFEAT, _NCLASS), lambda i: (0, 0)),
            pl.BlockSpec((_NCLASS, 1), lambda i: (0, 0)),
            pl.BlockSpec((_NCLASS, 1), lambda i: (0, 0)),
        ],
        out_specs=[
            pl.BlockSpec((_BR, _NCLASS), lambda i: (i, 0)),
            pl.BlockSpec((_BR, 16), lambda i: (i, 0)),
            pl.BlockSpec((_BR, 16), lambda i: (i, 0)),
        ],
        out_shape=[
            jax.ShapeDtypeStruct((_NP, _NCLASS), jnp.float32),
            jax.ShapeDtypeStruct((_NP, 16), jnp.float32),
            jax.ShapeDtypeStruct((_NP, 16), jnp.float32),
        ],
    )(zpart, dpart, w2, a2src, a2dst)


def _sc2_body(src_hbm, dst_hbm, ssrc_hbm, sdst_hbm, wh_hbm, zzero_hbm, dzero_hbm,
              zpart_hbm, dpart_hbm,
              srcv, dstv,
              sbuf0, dbuf0, pbuf0, fbuf0, sbuf1, dbuf1, pbuf1, fbuf1,
              zsh, dsh, gsem0, gsem1, csem0, csem1):
    cid = lax.axis_index("c")
    sid = lax.axis_index("s")
    wid = sid * _NC + cid
    r0 = sid * _ROWS

    pltpu.sync_copy(zzero_hbm.at[pl.ds(r0, _ROWS), :], zsh.at[pl.ds(r0, _ROWS), :])
    pltpu.sync_copy(dzero_hbm.at[pl.ds(r0, _ROWS), :], dsh.at[pl.ds(r0, _ROWS), :])

    pltpu.sync_copy(src_hbm.at[pl.ds(wid * _STEPS, _STEPS), :], srcv)
    pltpu.sync_copy(dst_hbm.at[pl.ds(wid * _STEPS, _STEPS), :], dstv)
    plsc.subcore_barrier()

    bufs = ((sbuf0, dbuf0, pbuf0, fbuf0, gsem0, csem0),
            (sbuf1, dbuf1, pbuf1, fbuf1, gsem1, csem1))

    def issue_gathers(s, b):
        sb, db, _, fb, gsem, _ = bufs[b]
        pltpu.async_copy(ssrc_hbm.at[srcv.at[s]], sb, gsem)
        pltpu.async_copy(sdst_hbm.at[dstv.at[s]], db, gsem)
        pltpu.async_copy(wh_hbm.at[srcv.at[s]], fb, gsem)

    def do_step(s, b):
        sb, db, pb, fb, gsem, csem = bufs[b]
        pltpu.make_async_copy(ssrc_hbm.at[srcv.at[s]], sb, gsem).wait()
        pltpu.make_async_copy(sdst_hbm.at[dstv.at[s]], db, gsem).wait()
        pltpu.make_async_copy(wh_hbm.at[srcv.at[s]], fb, gsem).wait()

        def fuse(e, c):
            t = sb[e, :] + db[e, :]
            t = jnp.maximum(t, t * _NEG)
            pv = jnp.exp(t)
            pb[e, :] = pv
            ph = pv[0]
            fb[e, pl.ds(0, 16)] = fb[e, pl.ds(0, 16)] * ph
            fb[e, pl.ds(16, 16)] = fb[e, pl.ds(16, 16)] * ph
            return c
        lax.fori_loop(0, _C, fuse, 0)

        c1 = pltpu.async_copy(pb, dsh.at[dstv.at[s]], csem, add=True)
        c2 = pltpu.async_copy(fb, zsh.at[dstv.at[s]], csem, add=True)
        c1.wait()
        c2.wait()

        @pl.when(s < _STEPS - 2)
        def _():
            issue_gathers(s + 2, b)

    issue_gathers(0, 0)
    issue_gathers(1, 1)

    def pair(i, carry):
        do_step(2 * i, 0)
        do_step(2 * i + 1, 1)
        return carry
    lax.fori_loop(0, _STEPS // 2, pair, 0)

    plsc.subcore_barrier()
    pltpu.sync_copy(zsh.at[pl.ds(r0, _ROWS), :], zpart_hbm.at[cid, pl.ds(r0, _ROWS), :])
    pltpu.sync_copy(dsh.at[pl.ds(r0, _ROWS), :], dpart_hbm.at[cid, pl.ds(r0, _ROWS), :])


def _sc2(src2d, dst2d, tabs2, tabd2, wh2, z2zero, d2zero):
    mesh = plsc.VectorSubcoreMesh(
        core_axis_name="c", subcore_axis_name="s", num_cores=_NC, num_subcores=_NS)
    f = functools.partial(
        pl.kernel,
        out_type=[
            jax.ShapeDtypeStruct((_NC, _NP, _NCLASS), jnp.float32),
            jax.ShapeDtypeStruct((_NC, _NP, 16), jnp.float32),
        ],
        mesh=mesh,
        scratch_types=[
            pltpu.VMEM((_STEPS, _C), jnp.int32),
            pltpu.VMEM((_STEPS, _C), jnp.int32),
            pltpu.VMEM((_C, 16), jnp.float32),
            pltpu.VMEM((_C, 16), jnp.float32),
            pltpu.VMEM((_C, 16), jnp.float32),
            pltpu.VMEM((_C, _NCLASS), jnp.float32),
            pltpu.VMEM((_C, 16), jnp.float32),
            pltpu.VMEM((_C, 16), jnp.float32),
            pltpu.VMEM((_C, 16), jnp.float32),
            pltpu.VMEM((_C, _NCLASS), jnp.float32),
            pltpu.VMEM_SHARED((_NP, _NCLASS), jnp.float32),
            pltpu.VMEM_SHARED((_NP, 16), jnp.float32),
            pltpu.SemaphoreType.DMA,
            pltpu.SemaphoreType.DMA,
            pltpu.SemaphoreType.DMA,
            pltpu.SemaphoreType.DMA,
        ],
        compiler_params=pltpu.CompilerParams(use_tc_tiling_on_sc=False),
    )(_sc2_body)
    return f(src2d, dst2d, tabs2, tabd2, wh2, z2zero, d2zero)


def _tc3_body(z_ref, d_ref, out_ref):
    z = z_ref[0] + z_ref[1]
    d = jnp.maximum(d_ref[0, :, :1] + d_ref[1, :, :1], 1e-16)
    out_ref[...] = z / d


def _tc3(z2part, d2part):
    return pl.pallas_call(
        _tc3_body,
        out_shape=jax.ShapeDtypeStruct((_NP, _NCLASS), jnp.float32),
    )(z2part, d2part)


def kernel(x, edge_index, W1, a1, W2, a2):
    w1cat = W1.transpose(1, 0, 2).reshape(_NFEAT, _NHEADS * _NHID)
    rows = jnp.arange(_NHEADS * _NHID)
    asrc = jnp.zeros((_NHEADS * _NHID, _NHEADS), jnp.float32).at[
        rows, rows // _NHID].set(a1[:, _NHID:].reshape(-1))
    adst = jnp.zeros((_NHEADS * _NHID, _NHEADS), jnp.float32).at[
        rows, rows // _NHID].set(a1[:, :_NHID].reshape(-1))
    a2src = a2[_NCLASS:].reshape(_NCLASS, 1)
    a2dst = a2[:_NCLASS].reshape(_NCLASS, 1)

    pad_idx = (_N + jnp.arange(_EP - _E, dtype=jnp.int32) % (_NP - _N))
    src2d = jnp.concatenate(
        [edge_index[0].astype(jnp.int32), pad_idx]).reshape(_EP // _C, _C)
    dst2d = jnp.concatenate(
        [edge_index[1].astype(jnp.int32), pad_idx]).reshape(_EP // _C, _C)

    zzero = jnp.zeros((_NP, _NFEAT), jnp.float32)
    dzero = jnp.zeros((_NP, 2 * _NHEADS), jnp.float32)
    z2zero = jnp.zeros((_NP, _NCLASS), jnp.float32)
    d2zero = jnp.zeros((_NP, 16), jnp.float32)

    wh1, tabs1, tabd1 = _tc1(x, w1cat, asrc, adst)
    zpart, dpart = _sc1(src2d, dst2d, tabs1, tabd1, wh1, zzero, dzero)
    wh2, tabs2, tabd2 = _tc2(zpart, dpart, W2, a2src, a2dst)
    z2part, d2part = _sc2(src2d, dst2d, tabs2, tabd2, wh2, z2zero, d2zero)
    out = _tc3(z2part, d2part)
    return out[:_N]
